# Initial kernel scaffold; baseline (speedup 1.0000x reference)
import jax
import jax.numpy as jnp
from jax.experimental import pallas as pl


def kernel(x, edge_index, node_index, Wl1, bl1, Wr1, Wl2, bl2, Wr2, W1, b1, W2, b2, W3, b3, g1, be1, g2, be2):
    raise NotImplementedError("write your pallas kernel here")


# speedup vs baseline: 7.5025x; 7.5025x over previous
import functools

import jax
import jax.numpy as jnp
from jax import lax
from jax.experimental import pallas as pl
from jax.experimental.pallas import tpu as pltpu
from jax.experimental.pallas import tpu_sc as plsc

_N = 10000
_E = 320000
_D = 128
_DA = 144
_B = 1024

_NC = 2
_NS = 16
_NW = _NC * _NS
_K = 80
_EPW = _E // _NW
_NCHUNK = _EPW // _K
_RPT = _N // _NS
_SPT = _B // _NS


def _mesh():
    return plsc.VectorSubcoreMesh(core_axis_name="c", subcore_axis_name="s",
                                  num_cores=_NC, num_subcores=_NS)


def _zero_vmem(ref, rows, cols):
    nlane = cols // 16

    def body(i, carry):
        r = i // nlane
        cidx = (i % nlane) * 16
        ref[r, pl.ds(cidx, 16)] = jnp.zeros((16,), jnp.float32)
        return carry

    lax.fori_loop(0, rows * nlane, body, 0)


def _sc_agg1(x_aug, src2d, dst2d):
    @functools.partial(
        pl.kernel,
        out_type=jax.ShapeDtypeStruct((_NC * _N, _DA), jnp.float32),
        mesh=_mesh(),
        compiler_params=pltpu.CompilerParams(use_tc_tiling_on_sc=False),
        scratch_types=[
            pltpu.VMEM_SHARED((_N, _DA), jnp.float32),
            pltpu.VMEM((_NCHUNK, _K), jnp.int32),
            pltpu.VMEM((_NCHUNK, _K), jnp.int32),
            pltpu.VMEM((_K, _DA), jnp.float32),
            pltpu.SemaphoreType.DMA,
        ],
    )
    def k(x_hbm, src_hbm, dst_hbm, out_hbm, shared, src_v, dst_v,
          rows_v, sem):
        c = lax.axis_index("c")
        s = lax.axis_index("s")
        w = c * _NS + s

        _zero_vmem(rows_v, _K, _DA)
        for z in range(_RPT // _K):
            pltpu.sync_copy(rows_v, shared.at[pl.ds(s * _RPT + z * _K, _K)])
        pltpu.sync_copy(rows_v.at[pl.ds(0, _RPT % _K)],
                        shared.at[pl.ds(s * _RPT + (_RPT // _K) * _K,
                                        _RPT % _K)])

        pltpu.sync_copy(src_hbm.at[pl.ds(w * _NCHUNK, _NCHUNK)], src_v)
        pltpu.sync_copy(dst_hbm.at[pl.ds(w * _NCHUNK, _NCHUNK)], dst_v)
        plsc.subcore_barrier()

        def body(j, carry):
            pltpu.async_copy(x_hbm.at[src_v.at[j]], rows_v, sem).wait()
            pltpu.sync_copy(rows_v, shared.at[dst_v.at[j]], add=True)
            return carry

        lax.fori_loop(0, _NCHUNK, body, 0)
        plsc.subcore_barrier()

        pltpu.sync_copy(shared.at[pl.ds(s * _RPT, _RPT)],
                        out_hbm.at[pl.ds(c * _N + s * _RPT, _RPT)])

    return k(x_aug, src2d, dst2d)


def _sc_agg2(h1, src2d, dst2d, node2d, cw):
    @functools.partial(
        pl.kernel,
        out_type=(
            jax.ShapeDtypeStruct((_NC * _B, _D), jnp.float32),
            jax.ShapeDtypeStruct((_B, _D), jnp.float32),
            jax.ShapeDtypeStruct((_B, _D), jnp.float32),
        ),
        mesh=_mesh(),
        compiler_params=pltpu.CompilerParams(use_tc_tiling_on_sc=False),
        scratch_types=[
            pltpu.VMEM_SHARED((_N, _D), jnp.float32),
            pltpu.VMEM((_NCHUNK, _K), jnp.int32),
            pltpu.VMEM((_NCHUNK, _K), jnp.int32),
            pltpu.VMEM((_K, _D), jnp.float32),
            pltpu.VMEM((_SPT,), jnp.int32),
            pltpu.VMEM((_SPT, _D), jnp.float32),
            pltpu.SemaphoreType.DMA,
        ],
    )
    def k(h1_hbm, src_hbm, dst_hbm, node_hbm, cw_hbm, sel_out, h1s_out,
          cws_out, shared, src_v, dst_v, rows_v, nidx_v, selrows_v, sem):
        c = lax.axis_index("c")
        s = lax.axis_index("s")
        w = c * _NS + s

        _zero_vmem(rows_v, _K, _D)
        for z in range(_RPT // _K):
            pltpu.sync_copy(rows_v, shared.at[pl.ds(s * _RPT + z * _K, _K)])
        pltpu.sync_copy(rows_v.at[pl.ds(0, _RPT % _K)],
                        shared.at[pl.ds(s * _RPT + (_RPT // _K) * _K,
                                        _RPT % _K)])

        pltpu.sync_copy(src_hbm.at[pl.ds(w * _NCHUNK, _NCHUNK)], src_v)
        pltpu.sync_copy(dst_hbm.at[pl.ds(w * _NCHUNK, _NCHUNK)], dst_v)
        pltpu.sync_copy(node_hbm.at[s], nidx_v)
        plsc.subcore_barrier()

        def body(j, carry):
            pltpu.async_copy(h1_hbm.at[src_v.at[j]], rows_v, sem).wait()
            pltpu.sync_copy(rows_v, shared.at[dst_v.at[j]], add=True)
            return carry

        lax.fori_loop(0, _NCHUNK, body, 0)
        plsc.subcore_barrier()

        pltpu.async_copy(shared.at[nidx_v], selrows_v, sem).wait()
        pltpu.sync_copy(selrows_v, sel_out.at[pl.ds(c * _B + s * _SPT, _SPT)])

        @pl.when(c == 0)
        def _():
            pltpu.async_copy(h1_hbm.at[nidx_v], selrows_v, sem).wait()
            pltpu.sync_copy(selrows_v, h1s_out.at[pl.ds(s * _SPT, _SPT)])

        @pl.when(c == 1)
        def _():
            pltpu.async_copy(cw_hbm.at[nidx_v], selrows_v, sem).wait()
            pltpu.sync_copy(selrows_v, cws_out.at[pl.ds(s * _SPT, _SPT)])

    return k(h1, src2d, dst2d, node2d, cw)


def _tc_layer1(parts, x, Wl1T, bl1r, Wr1T):
    blk = 1000
    grid = _N // blk

    def body(p0_ref, p1_ref, x_ref, wl_ref, bl_ref, wr_ref, h1_ref, cw_ref):
        ssum = p0_ref[...] + p1_ref[...]
        cnt = ssum[:, _D:_D + 1]
        inv = 1.0 / jnp.maximum(cnt, 1.0)
        mean = ssum[:, :_D] * inv
        h = (jnp.dot(mean, wl_ref[...], preferred_element_type=jnp.float32)
             + bl_ref[...]
             + jnp.dot(x_ref[...], wr_ref[...],
                       preferred_element_type=jnp.float32))
        h1_ref[...] = jnp.maximum(h, 0.0)
        cw_ref[...] = jnp.broadcast_to(inv, (blk, _D))

    return pl.pallas_call(
        body,
        grid=(grid,),
        in_specs=[
            pl.BlockSpec((blk, _DA), lambda i: (i, 0)),
            pl.BlockSpec((blk, _DA), lambda i: (i + grid, 0)),
            pl.BlockSpec((blk, _D), lambda i: (i, 0)),
            pl.BlockSpec((_D, _D), lambda i: (0, 0)),
            pl.BlockSpec((1, _D), lambda i: (0, 0)),
            pl.BlockSpec((_D, _D), lambda i: (0, 0)),
        ],
        out_specs=[
            pl.BlockSpec((blk, _D), lambda i: (i, 0)),
            pl.BlockSpec((blk, _D), lambda i: (i, 0)),
        ],
        out_shape=[
            jax.ShapeDtypeStruct((_N, _D), jnp.float32),
            jax.ShapeDtypeStruct((_N, _D), jnp.float32),
        ],
    )(parts, parts, x, Wl1T, bl1r, Wr1T)


def _tc_head(sel, h1s, cws, Wl2T, bl2r, Wr2T, W1T, b1r, W2T, b2r, W3T, b3r,
             g1r, be1r, g2r, be2r):
    def body(sel_ref, h1s_ref, cws_ref, wl_ref, bl_ref, wr_ref, w1_ref,
             b1_ref, w2_ref, b2_ref, w3_ref, b3_ref, g1_ref, be1_ref,
             g2_ref, be2_ref, out_ref):
        mean2 = (sel_ref[0:_B, :] + sel_ref[_B:2 * _B, :]) * cws_ref[...]
        h2 = (jnp.dot(mean2, wl_ref[...], preferred_element_type=jnp.float32)
              + bl_ref[...]
              + jnp.dot(h1s_ref[...], wr_ref[...],
                        preferred_element_type=jnp.float32))
        z = jnp.dot(h2, w1_ref[...], preferred_element_type=jnp.float32) \
            + b1_ref[...]
        mu = jnp.mean(z, axis=0, keepdims=True)
        var = jnp.mean((z - mu) ** 2, axis=0, keepdims=True)
        z = (z - mu) / jnp.sqrt(var + 1e-5) * g1_ref[...] + be1_ref[...]
        z = jnp.where(z > 0, z, 0.1 * z)
        z = jnp.dot(z, w2_ref[...], preferred_element_type=jnp.float32) \
            + b2_ref[...]
        mu = jnp.mean(z, axis=0, keepdims=True)
        var = jnp.mean((z - mu) ** 2, axis=0, keepdims=True)
        z = (z - mu) / jnp.sqrt(var + 1e-5) * g2_ref[...] + be2_ref[...]
        z = jnp.where(z > 0, z, 0.05 * z)
        out_ref[...] = jnp.dot(z, w3_ref[...],
                               preferred_element_type=jnp.float32) + b3_ref[...]

    return pl.pallas_call(
        body,
        out_shape=jax.ShapeDtypeStruct((_B, 1), jnp.float32),
    )(sel, h1s, cws, Wl2T, bl2r, Wr2T, W1T, b1r, W2T, b2r, W3T, b3r,
      g1r, be1r, g2r, be2r)


def kernel(x, edge_index, node_index, Wl1, bl1, Wr1, Wl2, bl2, Wr2,
           W1, b1, W2, b2, W3, b3, g1, be1, g2, be2):
    x_aug = jnp.concatenate(
        [x, jnp.ones((_N, 1), jnp.float32), jnp.zeros((_N, _DA - _D - 1),
                                                      jnp.float32)], axis=1)
    src2d = edge_index[0].reshape(_NW * _NCHUNK, _K)
    dst2d = edge_index[1].reshape(_NW * _NCHUNK, _K)
    node2d = node_index.reshape(_NS, _SPT)

    parts = _sc_agg1(x_aug, src2d, dst2d)
    h1, cw = _tc_layer1(parts, x, Wl1.T, bl1.reshape(1, -1), Wr1.T)
    sel, h1s, cws = _sc_agg2(h1, src2d, dst2d, node2d, cw)
    out = _tc_head(sel, h1s, cws, Wl2.T, bl2.reshape(1, -1), Wr2.T,
                   W1.T, b1.reshape(1, -1), W2.T, b2.reshape(1, -1),
                   W3.T, b3.reshape(1, -1), g1.reshape(1, -1),
                   be1.reshape(1, -1), g2.reshape(1, -1), be2.reshape(1, -1))
    return out
